# Initial kernel scaffold; baseline (speedup 1.0000x reference)
import jax
import jax.numpy as jnp
from jax.experimental import pallas as pl


def kernel(x, Wg, W1, W3, W2):
    raise NotImplementedError("write your pallas kernel here")


# speedup vs baseline: 1.7292x; 1.7292x over previous
import functools

import jax
import jax.numpy as jnp
from jax.experimental import pallas as pl
from jax.experimental.pallas import tpu as pltpu

N_EXP = 16
TOPK = 2


def _moe_body(x_ref, wgt_ref, w1_ref, w3_ref, w2_ref, y_ref, comb_ref):
    e = pl.program_id(0)

    @pl.when(e == 0)
    def _router():
        x = x_ref[...]
        logits = jnp.dot(x, wgt_ref[...], preferred_element_type=jnp.float32)
        m = jnp.max(logits, axis=-1, keepdims=True)
        p = jnp.exp(logits - m)
        p = p / jnp.sum(p, axis=-1, keepdims=True)
        idx = jax.lax.broadcasted_iota(jnp.int32, p.shape, 1)
        big = jnp.int32(N_EXP + 1)
        m1 = jnp.max(p, axis=-1, keepdims=True)
        i1 = jnp.min(jnp.where(p >= m1, idx, big), axis=-1, keepdims=True)
        pm = jnp.where(idx == i1, -jnp.inf, p)
        m2 = jnp.max(pm, axis=-1, keepdims=True)
        i2 = jnp.min(jnp.where(pm >= m2, idx, big), axis=-1, keepdims=True)
        comb_ref[...] = jnp.where(idx == i1, m1, 0.0) + jnp.where(idx == i2, m2, 0.0)

    x = x_ref[...]
    idx = jax.lax.broadcasted_iota(jnp.int32, (x.shape[0], N_EXP), 1)
    ce = jnp.sum(jnp.where(idx == e, comb_ref[...], 0.0), axis=-1, keepdims=True)
    h1 = jnp.dot(x, w1_ref[0], preferred_element_type=jnp.float32)
    h3 = jnp.dot(x, w3_ref[0], preferred_element_type=jnp.float32)
    h = (h1 / (1.0 + jnp.exp(-h1))) * h3
    out = jnp.dot(h, w2_ref[0], preferred_element_type=jnp.float32)

    @pl.when(e == 0)
    def _init():
        y_ref[...] = ce * out

    @pl.when(e > 0)
    def _acc():
        y_ref[...] += ce * out


@functools.partial(jax.jit, static_argnums=())
def kernel(x, Wg, W1, W3, W2):
    Bs, Ts, C = x.shape
    x_flat = x.reshape(-1, C)
    n = x_flat.shape[0]
    d_ff = W1.shape[-1]

    y = pl.pallas_call(
        _moe_body,
        grid=(N_EXP,),
        in_specs=[
            pl.BlockSpec((n, C), lambda e: (0, 0)),
            pl.BlockSpec((C, N_EXP), lambda e: (0, 0)),
            pl.BlockSpec((1, C, d_ff), lambda e: (e, 0, 0)),
            pl.BlockSpec((1, C, d_ff), lambda e: (e, 0, 0)),
            pl.BlockSpec((1, d_ff, C), lambda e: (e, 0, 0)),
        ],
        out_specs=pl.BlockSpec((n, C), lambda e: (0, 0)),
        out_shape=jax.ShapeDtypeStruct((n, C), jnp.float32),
        scratch_shapes=[pltpu.VMEM((n, N_EXP), jnp.float32)],
        compiler_params=pltpu.CompilerParams(
            dimension_semantics=("arbitrary",),
        ),
    )(x_flat, Wg.T, W1, W3, W2)
    return y.reshape(Bs, Ts, C)
